# Initial kernel scaffold; baseline (speedup 1.0000x reference)
import jax
import jax.numpy as jnp
from jax.experimental import pallas as pl


def kernel(X, edge_index, W1, b1, W2, b2, W_out, b_out):
    raise NotImplementedError("write your pallas kernel here")


# speedup vs baseline: 3.1883x; 3.1883x over previous
import jax
import jax.numpy as jnp
from jax import lax
from jax.experimental import pallas as pl
from jax.experimental.pallas import tpu as pltpu
from jax.experimental.pallas import tpu_sc as plsc

_LAM = 1.0
_ALPHA = 0.5
_STEPS = 16
_C = 128
_NT = 16


def _mlp_tc(Xp, W1, b1, W2, b2, NP, D, H, N, BR):
    G = NP // BR

    def body(x_ref, w1_ref, b1_ref, w2_ref, b2_ref, o_ref):
        x = x_ref[...]
        h = jnp.maximum(x @ w1_ref[...] + b1_ref[...], 0.0)
        y0 = h @ w2_ref[...] + b2_ref[...]
        i = pl.program_id(0)
        rows = i * BR + lax.broadcasted_iota(jnp.int32, (BR, 1), 0)
        y0 = jnp.where(rows < N, y0, 0.0)
        o_ref[0] = y0[:, :H]
        o_ref[1] = y0[:, H:]

    return pl.pallas_call(
        body,
        grid=(G,),
        in_specs=[
            pl.BlockSpec((BR, D), lambda i: (i, 0)),
            pl.BlockSpec((D, D), lambda i: (0, 0)),
            pl.BlockSpec((1, D), lambda i: (0, 0)),
            pl.BlockSpec((D, D), lambda i: (0, 0)),
            pl.BlockSpec((1, D), lambda i: (0, 0)),
        ],
        out_specs=pl.BlockSpec((2, BR, H), lambda i: (0, i, 0)),
        out_shape=jax.ShapeDtypeStruct((2, NP, H), jnp.float32),
    )(Xp, W1, b1.reshape(1, D), W2, b2.reshape(1, D))


def _head_tc(y_pair, W_out, b_out, NP, D, H, BR):
    G = NP // BR

    def body(y_ref, w_ref, b_ref, o_ref):
        w = w_ref[...]
        o_ref[...] = y_ref[0] @ w[:H] + y_ref[1] @ w[H:] + b_ref[...]

    return pl.pallas_call(
        body,
        grid=(G,),
        in_specs=[
            pl.BlockSpec((2, BR, H), lambda i: (0, i, 0)),
            pl.BlockSpec((D, D), lambda i: (0, 0)),
            pl.BlockSpec((1, D), lambda i: (0, 0)),
        ],
        out_specs=pl.BlockSpec((BR, D), lambda i: (i, 0)),
        out_shape=jax.ShapeDtypeStruct((NP, D), jnp.float32),
    )(y_pair, W_out, b_out.reshape(1, D))


def _sc_diffuse(y0_pair, srcp, dstp, NP, H, N, K):
    R = NP // _NT
    RB = 32
    RC = R // RB
    KG = K // 16
    HB = H // 16
    i32 = jnp.int32
    f32 = jnp.float32
    mesh = plsc.VectorSubcoreMesh(core_axis_name="c", subcore_axis_name="s")

    def body(y0_hbm, src_hbm, dst_hbm, out_hbm,
             acc_sp,
             sidx, didx, rows0, zb, accb, yb, y0b, degl, sv):
        c = lax.axis_index("c")
        t = lax.axis_index("s")
        row0 = t * R
        cofs = c * NP
        ones16 = jnp.ones((16,), f32)
        zeros16 = jnp.zeros((16,), f32)

        def zrow(i, _):
            for cb in range(HB):
                zb[i, pl.ds(cb * 16, 16)] = zeros16
            return 0
        lax.fori_loop(0, RB, zrow, 0)

        def zdeg(i, _):
            degl[pl.ds(i * 16, 16)] = zeros16
            return 0
        lax.fori_loop(0, R // 16, zdeg, 0)

        def dtile(u, _1):
            def dgroup(g, _2):
                pltpu.sync_copy(src_hbm.at[u, pl.ds(g * 16, 16)], sidx)

                def dchunk(j, _3):
                    for k in range(_C // 16):
                        idx = sidx[j, pl.ds(k * 16, 16)]
                        m = (idx >= row0) & (idx < row0 + R)
                        loc = jnp.where(m, idx - row0, 0)
                        plsc.addupdate_scatter(degl, [loc], ones16, mask=m)
                    return 0
                lax.fori_loop(0, 16, dchunk, 0)
                return 0
            lax.fori_loop(0, KG, dgroup, 0)
            return 0
        lax.fori_loop(0, _NT, dtile, 0)

        def srow(i, _):
            dk = degl[pl.ds(i * 16, 16)]
            sv[pl.ds(i * 16, 16)] = _ALPHA / (_LAM * dk + 1.0)
            return 0
        lax.fori_loop(0, R // 16, srow, 0)

        def init_chunk(rc, _):
            r0 = row0 + rc * RB
            pltpu.sync_copy(y0_hbm.at[c, pl.ds(r0, RB)], y0b)
            pltpu.sync_copy(y0b, out_hbm.at[pl.ds(cofs + r0, RB)])
            pltpu.sync_copy(zb, acc_sp.at[pl.ds(r0, RB)])
            return 0
        lax.fori_loop(0, RC, init_chunk, 0)
        plsc.subcore_barrier()

        def step(_s, carry):
            def group(g, _2):
                pltpu.sync_copy(src_hbm.at[t, pl.ds(g * 16, 16)], sidx)
                pltpu.sync_copy(dst_hbm.at[t, pl.ds(g * 16, 16)], didx)

                def ofs(j, _3):
                    for k in range(_C // 16):
                        sl = pl.ds(k * 16, 16)
                        didx[j, sl] = didx[j, sl] + cofs
                    return 0
                lax.fori_loop(0, 16, ofs, 0)

                def chunk(j, _3):
                    pltpu.sync_copy(out_hbm.at[didx.at[j]], rows0)
                    pltpu.sync_copy(rows0, acc_sp.at[sidx.at[j]], add=True)
                    return 0
                lax.fori_loop(0, 16, chunk, 0)
                return 0
            lax.fori_loop(0, KG, group, 0)
            plsc.subcore_barrier()

            def upd_chunk(rc, _2):
                r0 = row0 + rc * RB
                pltpu.sync_copy(acc_sp.at[pl.ds(r0, RB)], accb)
                pltpu.sync_copy(out_hbm.at[pl.ds(cofs + r0, RB)], yb)
                pltpu.sync_copy(y0_hbm.at[c, pl.ds(r0, RB)], y0b)

                def urow(r, _3):
                    sbc = plsc.load_gather(
                        sv, [jnp.full((16,), rc * RB + r, i32)])
                    for cb in range(HB):
                        sl = pl.ds(cb * 16, 16)
                        a = accb[r, sl]
                        y = yb[r, sl]
                        y0v = y0b[r, sl]
                        yb[r, sl] = (1.0 - _ALPHA) * y + sbc * (_LAM * a + y0v)
                    return 0
                lax.fori_loop(0, RB, urow, 0)
                pltpu.sync_copy(yb, out_hbm.at[pl.ds(cofs + r0, RB)])
                pltpu.sync_copy(zb, acc_sp.at[pl.ds(r0, RB)])
                return 0
            lax.fori_loop(0, RC, upd_chunk, 0)
            plsc.subcore_barrier()
            return carry
        lax.fori_loop(0, _STEPS, step, 0)

    fn = pl.kernel(
        body,
        out_type=jax.ShapeDtypeStruct((2 * NP, H), f32),
        mesh=mesh,
        compiler_params=pltpu.CompilerParams(
            needs_layout_passes=False, use_tc_tiling_on_sc=False),
        scratch_types=[
            pltpu.VMEM_SHARED((2 * NP, H), f32),
            pltpu.VMEM((16, _C), i32),
            pltpu.VMEM((16, _C), i32),
            pltpu.VMEM((_C, H), f32),
            pltpu.VMEM((RB, H), f32),
            pltpu.VMEM((RB, H), f32),
            pltpu.VMEM((RB, H), f32),
            pltpu.VMEM((RB, H), f32),
            pltpu.VMEM((R,), f32),
            pltpu.VMEM((R,), f32),
        ],
    )
    return fn(y0_pair, srcp, dstp).reshape(2, NP, H)


def kernel(X, edge_index, W1, b1, W2, b2, W_out, b_out):
    N, D = X.shape
    H = D // 2
    E = edge_index.shape[1]
    NP = ((N + 1 + 2047) // 2048) * 2048
    NCH = -(-E // _C)
    K = 16 * (-(-NCH // (_NT * 16)))
    PADE = _NT * K * _C

    src = edge_index[0]
    dst = edge_index[1]
    pad = jnp.full((PADE - E,), N, jnp.int32)
    srcp = jnp.concatenate([src, pad]).reshape(K, _NT, _C).transpose(1, 0, 2)
    dstp = jnp.concatenate([dst, pad]).reshape(K, _NT, _C).transpose(1, 0, 2)
    Xp = jnp.pad(X, ((0, NP - N), (0, 0)))

    BR = NP // 16
    y0_pair = _mlp_tc(Xp, W1, b1, W2, b2, NP, D, H, N, BR)
    y_pair = _sc_diffuse(y0_pair, srcp, dstp, NP, H, N, K)
    out = _head_tc(y_pair, W_out, b_out, NP, D, H, BR)
    return out[:N]


# speedup vs baseline: 3.7429x; 1.1740x over previous
import jax
import jax.numpy as jnp
from jax import lax
from jax.experimental import pallas as pl
from jax.experimental.pallas import tpu as pltpu
from jax.experimental.pallas import tpu_sc as plsc

_LAM = 1.0
_ALPHA = 0.5
_STEPS = 16
_C = 128
_NT = 16


def _mlp_tc(Xp, W1, b1, W2, b2, NP, D, H, N, BR):
    G = NP // BR

    def body(x_ref, w1_ref, b1_ref, w2_ref, b2_ref, o_ref):
        x = x_ref[...]
        h = jnp.maximum(x @ w1_ref[...] + b1_ref[...], 0.0)
        y0 = h @ w2_ref[...] + b2_ref[...]
        i = pl.program_id(0)
        rows = i * BR + lax.broadcasted_iota(jnp.int32, (BR, 1), 0)
        y0 = jnp.where(rows < N, y0, 0.0)
        o_ref[0] = y0[:, :H]
        o_ref[1] = y0[:, H:]

    return pl.pallas_call(
        body,
        grid=(G,),
        in_specs=[
            pl.BlockSpec((BR, D), lambda i: (i, 0)),
            pl.BlockSpec((D, D), lambda i: (0, 0)),
            pl.BlockSpec((1, D), lambda i: (0, 0)),
            pl.BlockSpec((D, D), lambda i: (0, 0)),
            pl.BlockSpec((1, D), lambda i: (0, 0)),
        ],
        out_specs=pl.BlockSpec((2, BR, H), lambda i: (0, i, 0)),
        out_shape=jax.ShapeDtypeStruct((2, NP, H), jnp.float32),
    )(Xp, W1, b1.reshape(1, D), W2, b2.reshape(1, D))


def _head_tc(y_pair, W_out, b_out, NP, D, H, BR):
    G = NP // BR

    def body(y_ref, w_ref, b_ref, o_ref):
        w = w_ref[...]
        o_ref[...] = y_ref[0] @ w[:H] + y_ref[1] @ w[H:] + b_ref[...]

    return pl.pallas_call(
        body,
        grid=(G,),
        in_specs=[
            pl.BlockSpec((2, BR, H), lambda i: (0, i, 0)),
            pl.BlockSpec((D, D), lambda i: (0, 0)),
            pl.BlockSpec((1, D), lambda i: (0, 0)),
        ],
        out_specs=pl.BlockSpec((BR, D), lambda i: (i, 0)),
        out_shape=jax.ShapeDtypeStruct((NP, D), jnp.float32),
    )(y_pair, W_out, b_out.reshape(1, D))


def _sc_diffuse(y0_pair, srcp, dstp2, NP, H, N, K):
    R = NP // _NT
    RB = 64
    RC = R // RB
    KG = K // 16
    HB = H // 16
    i32 = jnp.int32
    f32 = jnp.float32
    mesh = plsc.VectorSubcoreMesh(core_axis_name="c", subcore_axis_name="s")

    def body(y0_hbm, src_hbm, dst_hbm, out_hbm,
             acc_sp,
             sidx, didx, rows0, rows1, accb, yb, y0b, degl, sv,
             semg0, semg1, sems0, sems1, semis, semid,
             semba, semby, semb0, semw0, semw1):
        c = lax.axis_index("c")
        t = lax.axis_index("s")
        row0 = t * R
        cofs = c * NP
        ones16 = jnp.ones((16,), f32)
        zeros16 = jnp.zeros((16,), f32)
        rows = (rows0, rows1)
        semg = (semg0, semg1)
        sems = (sems0, sems1)

        def zdeg(i, _):
            degl[pl.ds(i * 16, 16)] = zeros16
            return 0
        lax.fori_loop(0, R // 16, zdeg, 0)

        def dtile(u, _1):
            def dgroup(g, _2):
                pltpu.sync_copy(src_hbm.at[u, pl.ds(g * 16, 16)], sidx.at[0])

                def dchunk(j, _3):
                    for k in range(_C // 16):
                        idx = sidx[0, j, pl.ds(k * 16, 16)]
                        m = (idx >= row0) & (idx < row0 + R)
                        loc = jnp.where(m, idx - row0, 0)
                        plsc.addupdate_scatter(degl, [loc], ones16, mask=m)
                    return 0
                lax.fori_loop(0, 16, dchunk, 0)
                return 0
            lax.fori_loop(0, KG, dgroup, 0)
            return 0
        lax.fori_loop(0, _NT, dtile, 0)

        def srow(i, _):
            dk = degl[pl.ds(i * 16, 16)]
            sv[pl.ds(i * 16, 16)] = _ALPHA / (_LAM * dk + 1.0)
            return 0
        lax.fori_loop(0, R // 16, srow, 0)

        def zacc(i, _):
            for cb in range(HB):
                accb[i, pl.ds(cb * 16, 16)] = zeros16
            return 0
        lax.fori_loop(0, RB, zacc, 0)

        def init_chunk(rc, _):
            r0 = row0 + rc * RB
            pltpu.sync_copy(y0_hbm.at[c, pl.ds(r0, RB)], y0b)
            pltpu.sync_copy(y0b, out_hbm.at[pl.ds(cofs + r0, RB)])
            pltpu.sync_copy(accb, acc_sp.at[pl.ds(r0, RB)])
            return 0
        lax.fori_loop(0, RC, init_chunk, 0)
        plsc.subcore_barrier()

        def step(_s, carry):
            pltpu.sync_copy(src_hbm.at[t, pl.ds(0, 16)], sidx.at[0])
            pltpu.sync_copy(dst_hbm.at[c, t, pl.ds(0, 16)], didx.at[0])
            pltpu.async_copy(out_hbm.at[didx.at[0, 0]], rows0, semg0)

            def pair(j2, _2):
                for p in range(2):
                    j = 2 * j2 + p
                    gg = j // 16
                    jj = j - gg * 16
                    slot = jnp.bitwise_and(gg, 1)
                    pltpu.make_async_copy(
                        out_hbm.at[didx.at[slot, jj]], rows[p], semg[p]).wait()
                    pltpu.async_copy(
                        rows[p], acc_sp.at[sidx.at[slot, jj]], sems[p],
                        add=True)
                    @pl.when(jnp.logical_and(jj == 8, gg + 1 < KG))
                    def _():
                        nslot = jnp.bitwise_xor(slot, 1)
                        g1 = (gg + 1) * 16
                        pltpu.async_copy(
                            src_hbm.at[t, pl.ds(g1, 16)], sidx.at[nslot],
                            semis)
                        pltpu.async_copy(
                            dst_hbm.at[c, t, pl.ds(g1, 16)], didx.at[nslot],
                            semid)
                    @pl.when(jnp.logical_and(jj == 15, gg + 1 < KG))
                    def _():
                        nslot = jnp.bitwise_xor(slot, 1)
                        g1 = (gg + 1) * 16
                        pltpu.make_async_copy(
                            src_hbm.at[t, pl.ds(g1, 16)], sidx.at[nslot],
                            semis).wait()
                        pltpu.make_async_copy(
                            dst_hbm.at[c, t, pl.ds(g1, 16)], didx.at[nslot],
                            semid).wait()
                    @pl.when(j > 0)
                    def _():
                        pltpu.make_async_copy(
                            rows[p ^ 1], acc_sp.at[sidx.at[slot, jj]],
                            sems[p ^ 1]).wait()
                    @pl.when(j + 1 < K)
                    def _():
                        g1 = j + 1
                        gg1 = g1 // 16
                        jj1 = g1 - gg1 * 16
                        slot1 = jnp.bitwise_and(gg1, 1)
                        pltpu.async_copy(
                            out_hbm.at[didx.at[slot1, jj1]], rows[p ^ 1],
                            semg[p ^ 1])
                return 0
            lax.fori_loop(0, K // 2, pair, 0)
            pltpu.make_async_copy(
                rows[1], acc_sp.at[sidx.at[jnp.bitwise_and(KG - 1, 1), 15]],
                sems[1]).wait()
            plsc.subcore_barrier()

            def upd_chunk(rc, _2):
                r0 = row0 + rc * RB
                @pl.when(rc > 0)
                def _():
                    rp = row0 + (rc - 1) * RB
                    pltpu.make_async_copy(
                        yb, out_hbm.at[pl.ds(cofs + rp, RB)], semw0).wait()
                    pltpu.make_async_copy(
                        accb, acc_sp.at[pl.ds(rp, RB)], semw1).wait()
                pltpu.async_copy(acc_sp.at[pl.ds(r0, RB)], accb, semba)
                pltpu.async_copy(
                    out_hbm.at[pl.ds(cofs + r0, RB)], yb, semby)
                pltpu.async_copy(y0_hbm.at[c, pl.ds(r0, RB)], y0b, semb0)
                pltpu.make_async_copy(
                    acc_sp.at[pl.ds(r0, RB)], accb, semba).wait()
                pltpu.make_async_copy(
                    out_hbm.at[pl.ds(cofs + r0, RB)], yb, semby).wait()
                pltpu.make_async_copy(
                    y0_hbm.at[c, pl.ds(r0, RB)], y0b, semb0).wait()

                def urow(r, _3):
                    sbc = plsc.load_gather(
                        sv, [jnp.full((16,), rc * RB + r, i32)])
                    for cb in range(HB):
                        sl = pl.ds(cb * 16, 16)
                        a = accb[r, sl]
                        y = yb[r, sl]
                        y0v = y0b[r, sl]
                        yb[r, sl] = (1.0 - _ALPHA) * y + sbc * (_LAM * a + y0v)
                        accb[r, sl] = zeros16
                    return 0
                lax.fori_loop(0, RB, urow, 0)
                pltpu.async_copy(
                    yb, out_hbm.at[pl.ds(cofs + r0, RB)], semw0)
                pltpu.async_copy(accb, acc_sp.at[pl.ds(r0, RB)], semw1)
                return 0
            lax.fori_loop(0, RC, upd_chunk, 0)
            rl = row0 + (RC - 1) * RB
            pltpu.make_async_copy(
                yb, out_hbm.at[pl.ds(cofs + rl, RB)], semw0).wait()
            pltpu.make_async_copy(
                accb, acc_sp.at[pl.ds(rl, RB)], semw1).wait()
            plsc.subcore_barrier()
            return carry
        lax.fori_loop(0, _STEPS, step, 0)

    fn = pl.kernel(
        body,
        out_type=jax.ShapeDtypeStruct((2 * NP, H), f32),
        mesh=mesh,
        compiler_params=pltpu.CompilerParams(
            needs_layout_passes=False, use_tc_tiling_on_sc=False),
        scratch_types=[
            pltpu.VMEM_SHARED((2 * NP, H), f32),
            pltpu.VMEM((2, 16, _C), i32),
            pltpu.VMEM((2, 16, _C), i32),
            pltpu.VMEM((_C, H), f32),
            pltpu.VMEM((_C, H), f32),
            pltpu.VMEM((RB, H), f32),
            pltpu.VMEM((RB, H), f32),
            pltpu.VMEM((RB, H), f32),
            pltpu.VMEM((R,), f32),
            pltpu.VMEM((R,), f32),
            pltpu.SemaphoreType.DMA,
            pltpu.SemaphoreType.DMA,
            pltpu.SemaphoreType.DMA,
            pltpu.SemaphoreType.DMA,
            pltpu.SemaphoreType.DMA,
            pltpu.SemaphoreType.DMA,
            pltpu.SemaphoreType.DMA,
            pltpu.SemaphoreType.DMA,
            pltpu.SemaphoreType.DMA,
            pltpu.SemaphoreType.DMA,
            pltpu.SemaphoreType.DMA,
        ],
    )
    return fn(y0_pair, srcp, dstp2).reshape(2, NP, H)


def kernel(X, edge_index, W1, b1, W2, b2, W_out, b_out):
    N, D = X.shape
    H = D // 2
    E = edge_index.shape[1]
    NP = ((N + 1 + 2047) // 2048) * 2048
    NCH = -(-E // _C)
    K = 16 * (-(-NCH // (_NT * 16)))
    PADE = _NT * K * _C

    src = edge_index[0]
    dst = edge_index[1]
    pad = jnp.full((PADE - E,), N, jnp.int32)
    srcp = jnp.concatenate([src, pad]).reshape(K, _NT, _C).transpose(1, 0, 2)
    dstp = jnp.concatenate([dst, pad]).reshape(K, _NT, _C).transpose(1, 0, 2)
    dstp2 = jnp.stack([dstp, dstp + NP])
    Xp = jnp.pad(X, ((0, NP - N), (0, 0)))

    BR = NP // 16
    y0_pair = _mlp_tc(Xp, W1, b1, W2, b2, NP, D, H, N, BR)
    y_pair = _sc_diffuse(y0_pair, srcp, dstp2, NP, H, N, K)
    out = _head_tc(y_pair, W_out, b_out, NP, D, H, BR)
    return out[:N]


# speedup vs baseline: 4.0779x; 1.0895x over previous
import jax
import jax.numpy as jnp
from jax import lax
from jax.experimental import pallas as pl
from jax.experimental.pallas import tpu as pltpu
from jax.experimental.pallas import tpu_sc as plsc

_LAM = 1.0
_ALPHA = 0.5
_STEPS = 16
_C = 128
_NT = 16


def _mlp_tc(Xp, W1, b1, W2, b2, NP, D, H, N, BR):
    G = NP // BR

    def body(x_ref, w1_ref, b1_ref, w2_ref, b2_ref, o_ref):
        x = x_ref[...]
        h = jnp.maximum(x @ w1_ref[...] + b1_ref[...], 0.0)
        y0 = h @ w2_ref[...] + b2_ref[...]
        i = pl.program_id(0)
        rows = i * BR + lax.broadcasted_iota(jnp.int32, (BR, 1), 0)
        y0 = jnp.where(rows < N, y0, 0.0)
        o_ref[0] = y0[:, :H]
        o_ref[1] = y0[:, H:]

    return pl.pallas_call(
        body,
        grid=(G,),
        in_specs=[
            pl.BlockSpec((BR, D), lambda i: (i, 0)),
            pl.BlockSpec((D, D), lambda i: (0, 0)),
            pl.BlockSpec((1, D), lambda i: (0, 0)),
            pl.BlockSpec((D, D), lambda i: (0, 0)),
            pl.BlockSpec((1, D), lambda i: (0, 0)),
        ],
        out_specs=pl.BlockSpec((2, BR, H), lambda i: (0, i, 0)),
        out_shape=jax.ShapeDtypeStruct((2, NP, H), jnp.float32),
    )(Xp, W1, b1.reshape(1, D), W2, b2.reshape(1, D))


def _head_tc(y_pair, W_out, b_out, NP, D, H, BR):
    G = NP // BR

    def body(y_ref, w_ref, b_ref, o_ref):
        w = w_ref[...]
        o_ref[...] = y_ref[0] @ w[:H] + y_ref[1] @ w[H:] + b_ref[...]

    return pl.pallas_call(
        body,
        grid=(G,),
        in_specs=[
            pl.BlockSpec((2, BR, H), lambda i: (0, i, 0)),
            pl.BlockSpec((D, D), lambda i: (0, 0)),
            pl.BlockSpec((1, D), lambda i: (0, 0)),
        ],
        out_specs=pl.BlockSpec((BR, D), lambda i: (i, 0)),
        out_shape=jax.ShapeDtypeStruct((NP, D), jnp.float32),
    )(y_pair, W_out, b_out.reshape(1, D))


def _sc_diffuse(y0_pair, srcp, dstp2, NP, H, N, K):
    R = NP // _NT
    RB = 80
    RC = R // RB
    KG = K // 16
    HB = H // 16
    i32 = jnp.int32
    f32 = jnp.float32
    mesh = plsc.VectorSubcoreMesh(core_axis_name="c", subcore_axis_name="s")

    def body(y0_hbm, src_hbm, dst_hbm, out_hbm,
             acc_sp, deg_sp,
             sidx, didx, rows0, rows1, accb, yb, y0b, degl, sv, onesb,
             semg0, semg1, sems0, sems1, semis, semid,
             semba, semby, semb0, semw0, semw1):
        c = lax.axis_index("c")
        t = lax.axis_index("s")
        row0 = t * R
        cofs = c * NP
        ones16 = jnp.ones((16,), f32)
        zeros16 = jnp.zeros((16,), f32)
        rows = (rows0, rows1)
        semg = (semg0, semg1)
        sems = (sems0, sems1)

        def zdeg(i, _):
            degl[pl.ds(i * 16, 16)] = zeros16
            return 0
        lax.fori_loop(0, R // 16, zdeg, 0)
        for k in range(_C // 16):
            onesb[pl.ds(k * 16, 16)] = ones16

        pltpu.sync_copy(degl, deg_sp.at[pl.ds(row0, R)])
        plsc.subcore_barrier()

        def dgroup(g, _2):
            pltpu.sync_copy(src_hbm.at[t, pl.ds(g * 16, 16)], sidx.at[0])

            def dchunk(j, _3):
                pltpu.sync_copy(onesb, deg_sp.at[sidx.at[0, j]], add=True)
                return 0
            lax.fori_loop(0, 16, dchunk, 0)
            return 0
        lax.fori_loop(0, KG, dgroup, 0)
        plsc.subcore_barrier()

        pltpu.sync_copy(deg_sp.at[pl.ds(row0, R)], degl)

        def srow(i, _):
            dk = degl[pl.ds(i * 16, 16)]
            sv[pl.ds(i * 16, 16)] = _ALPHA / (_LAM * dk + 1.0)
            return 0
        lax.fori_loop(0, R // 16, srow, 0)

        def zacc(i, _):
            for cb in range(HB):
                accb[i, pl.ds(cb * 16, 16)] = zeros16
            return 0
        lax.fori_loop(0, RB, zacc, 0)

        def init_chunk(rc, _):
            r0 = row0 + rc * RB
            pltpu.sync_copy(y0_hbm.at[c, pl.ds(r0, RB)], y0b)
            pltpu.sync_copy(y0b, out_hbm.at[pl.ds(cofs + r0, RB)])
            pltpu.sync_copy(accb, acc_sp.at[pl.ds(r0, RB)])
            return 0
        lax.fori_loop(0, RC, init_chunk, 0)
        plsc.subcore_barrier()

        def step(_s, carry):
            pltpu.sync_copy(src_hbm.at[t, pl.ds(0, 16)], sidx.at[0])
            pltpu.sync_copy(dst_hbm.at[c, t, pl.ds(0, 16)], didx.at[0])
            pltpu.async_copy(out_hbm.at[didx.at[0, 0]], rows0, semg0)

            def pair(j2, _2):
                for p in range(2):
                    j = 2 * j2 + p
                    gg = j // 16
                    jj = j - gg * 16
                    slot = jnp.bitwise_and(gg, 1)
                    pltpu.make_async_copy(
                        out_hbm.at[didx.at[slot, jj]], rows[p], semg[p]).wait()
                    pltpu.async_copy(
                        rows[p], acc_sp.at[sidx.at[slot, jj]], sems[p],
                        add=True)
                    @pl.when(jnp.logical_and(jj == 8, gg + 1 < KG))
                    def _():
                        nslot = jnp.bitwise_xor(slot, 1)
                        g1 = (gg + 1) * 16
                        pltpu.async_copy(
                            src_hbm.at[t, pl.ds(g1, 16)], sidx.at[nslot],
                            semis)
                        pltpu.async_copy(
                            dst_hbm.at[c, t, pl.ds(g1, 16)], didx.at[nslot],
                            semid)
                    @pl.when(jnp.logical_and(jj == 15, gg + 1 < KG))
                    def _():
                        nslot = jnp.bitwise_xor(slot, 1)
                        g1 = (gg + 1) * 16
                        pltpu.make_async_copy(
                            src_hbm.at[t, pl.ds(g1, 16)], sidx.at[nslot],
                            semis).wait()
                        pltpu.make_async_copy(
                            dst_hbm.at[c, t, pl.ds(g1, 16)], didx.at[nslot],
                            semid).wait()
                    @pl.when(j > 0)
                    def _():
                        pltpu.make_async_copy(
                            rows[p ^ 1], acc_sp.at[sidx.at[slot, jj]],
                            sems[p ^ 1]).wait()
                    @pl.when(j + 1 < K)
                    def _():
                        g1 = j + 1
                        gg1 = g1 // 16
                        jj1 = g1 - gg1 * 16
                        slot1 = jnp.bitwise_and(gg1, 1)
                        pltpu.async_copy(
                            out_hbm.at[didx.at[slot1, jj1]], rows[p ^ 1],
                            semg[p ^ 1])
                return 0
            lax.fori_loop(0, K // 2, pair, 0)
            pltpu.make_async_copy(
                rows[1], acc_sp.at[sidx.at[jnp.bitwise_and(KG - 1, 1), 15]],
                sems[1]).wait()
            plsc.subcore_barrier()

            def upd_chunk(rc, _2):
                r0 = row0 + rc * RB
                @pl.when(rc > 0)
                def _():
                    rp = row0 + (rc - 1) * RB
                    pltpu.make_async_copy(
                        yb, out_hbm.at[pl.ds(cofs + rp, RB)], semw0).wait()
                    pltpu.make_async_copy(
                        accb, acc_sp.at[pl.ds(rp, RB)], semw1).wait()
                pltpu.async_copy(acc_sp.at[pl.ds(r0, RB)], accb, semba)
                pltpu.async_copy(
                    out_hbm.at[pl.ds(cofs + r0, RB)], yb, semby)
                pltpu.async_copy(y0_hbm.at[c, pl.ds(r0, RB)], y0b, semb0)
                pltpu.make_async_copy(
                    acc_sp.at[pl.ds(r0, RB)], accb, semba).wait()
                pltpu.make_async_copy(
                    out_hbm.at[pl.ds(cofs + r0, RB)], yb, semby).wait()
                pltpu.make_async_copy(
                    y0_hbm.at[c, pl.ds(r0, RB)], y0b, semb0).wait()

                def urow(r, _3):
                    sbc = plsc.load_gather(
                        sv, [jnp.full((16,), rc * RB + r, i32)])
                    for cb in range(HB):
                        sl = pl.ds(cb * 16, 16)
                        a = accb[r, sl]
                        y = yb[r, sl]
                        y0v = y0b[r, sl]
                        yb[r, sl] = (1.0 - _ALPHA) * y + sbc * (_LAM * a + y0v)
                        accb[r, sl] = zeros16
                    return 0
                lax.fori_loop(0, RB, urow, 0)
                pltpu.async_copy(
                    yb, out_hbm.at[pl.ds(cofs + r0, RB)], semw0)
                pltpu.async_copy(accb, acc_sp.at[pl.ds(r0, RB)], semw1)
                return 0
            lax.fori_loop(0, RC, upd_chunk, 0)
            rl = row0 + (RC - 1) * RB
            pltpu.make_async_copy(
                yb, out_hbm.at[pl.ds(cofs + rl, RB)], semw0).wait()
            pltpu.make_async_copy(
                accb, acc_sp.at[pl.ds(rl, RB)], semw1).wait()
            plsc.subcore_barrier()
            return carry
        lax.fori_loop(0, _STEPS, step, 0)

    fn = pl.kernel(
        body,
        out_type=jax.ShapeDtypeStruct((2 * NP, H), f32),
        mesh=mesh,
        compiler_params=pltpu.CompilerParams(
            needs_layout_passes=False, use_tc_tiling_on_sc=False),
        scratch_types=[
            pltpu.VMEM_SHARED((2 * NP, H), f32),
            pltpu.VMEM_SHARED((2 * NP,), f32),
            pltpu.VMEM((2, 16, _C), i32),
            pltpu.VMEM((2, 16, _C), i32),
            pltpu.VMEM((_C, H), f32),
            pltpu.VMEM((_C, H), f32),
            pltpu.VMEM((RB, H), f32),
            pltpu.VMEM((RB, H), f32),
            pltpu.VMEM((RB, H), f32),
            pltpu.VMEM((R,), f32),
            pltpu.VMEM((R,), f32),
            pltpu.VMEM((_C,), f32),
            pltpu.SemaphoreType.DMA,
            pltpu.SemaphoreType.DMA,
            pltpu.SemaphoreType.DMA,
            pltpu.SemaphoreType.DMA,
            pltpu.SemaphoreType.DMA,
            pltpu.SemaphoreType.DMA,
            pltpu.SemaphoreType.DMA,
            pltpu.SemaphoreType.DMA,
            pltpu.SemaphoreType.DMA,
            pltpu.SemaphoreType.DMA,
            pltpu.SemaphoreType.DMA,
        ],
    )
    return fn(y0_pair, srcp, dstp2).reshape(2, NP, H)


def kernel(X, edge_index, W1, b1, W2, b2, W_out, b_out):
    N, D = X.shape
    H = D // 2
    E = edge_index.shape[1]
    NP = ((N + 1 + 2047) // 2048) * 2048
    NCH = -(-E // _C)
    K = 16 * (-(-NCH // (_NT * 16)))
    PADE = _NT * K * _C

    src = edge_index[0]
    dst = edge_index[1]
    pad = jnp.full((PADE - E,), N, jnp.int32)
    srcp = jnp.concatenate([src, pad]).reshape(K, _NT, _C).transpose(1, 0, 2)
    dstp = jnp.concatenate([dst, pad]).reshape(K, _NT, _C).transpose(1, 0, 2)
    dstp2 = jnp.stack([dstp, dstp + NP])
    Xp = jnp.pad(X, ((0, NP - N), (0, 0)))

    BR = NP // 16
    y0_pair = _mlp_tc(Xp, W1, b1, W2, b2, NP, D, H, N, BR)
    y_pair = _sc_diffuse(y0_pair, srcp, dstp2, NP, H, N, K)
    out = _head_tc(y_pair, W_out, b_out, NP, D, H, BR)
    return out[:N]
